# Initial kernel scaffold; baseline (speedup 1.0000x reference)
import jax
import jax.numpy as jnp
from jax.experimental import pallas as pl


def kernel(x, ln1_g, ln2_g, Wqkv, Wo, Wg, W1, W2):
    raise NotImplementedError("write your pallas kernel here")


# speedup vs baseline: 1.2938x; 1.2938x over previous
import functools

import jax
import jax.numpy as jnp
import numpy as np
from jax.experimental import pallas as pl
from jax.experimental.pallas import tpu as pltpu

S, D, H, E = 2048, 768, 12, 64
DH = D // H
FF = 2 * D
TOPK = 1
CAP = int(1.25 * S / E)
EPS = 1e-6
ZW = 1e-3
NEG = -1e9
HALF = DH // 2

SBLK = 256
NS = S // SBLK
NPAD = ((E * CAP + 8) // 8) * 8

HIGH = jax.lax.Precision.HIGHEST


def _dot(a, b, dims):
    return jax.lax.dot_general(a.astype(jnp.bfloat16), b.astype(jnp.bfloat16),
                               dims, preferred_element_type=jnp.float32)


def _rms(x, g):
    return x * jax.lax.rsqrt(jnp.mean(x * x, axis=-1, keepdims=True) + EPS) * g


def _rope_swap(x):
    parts = []
    for h in range(H):
        parts.append(x[:, h * DH + HALF:h * DH + DH])
        parts.append(x[:, h * DH:h * DH + HALF])
    return jnp.concatenate(parts, axis=1)


def _k1_qkv(x_ref, g_ref, w_ref, cos_ref, sin_ref, q_ref, k_ref, v_ref):
    xn = _rms(x_ref[...], g_ref[...])
    qkv = _dot(xn, w_ref[...], (((1,), (0,)), ((), ())))
    q, k, v = qkv[:, :D], qkv[:, D:2 * D], qkv[:, 2 * D:]
    c, s = cos_ref[...], sin_ref[...]
    q_ref[...] = q * c + _rope_swap(q) * s
    k_ref[...] = k * c + _rope_swap(k) * s
    v_ref[...] = v


def _k2_attn(q_ref, k_ref, v_ref, o_ref):
    qi = pl.program_id(0)
    row = jax.lax.broadcasted_iota(jnp.int32, (SBLK, S), 0) + qi * SBLK
    col = jax.lax.broadcasted_iota(jnp.int32, (SBLK, S), 1)
    causal = col <= row
    for h in range(H):
        sl = slice(h * DH, (h + 1) * DH)
        sc = _dot(q_ref[:, sl], k_ref[:, sl],
                  (((1,), (1,)), ((), ()))) * (1.0 / np.sqrt(DH))
        sc = jnp.where(causal, sc, NEG)
        m = jnp.max(sc, axis=1, keepdims=True)
        p = jnp.exp(sc - m)
        a = p / jnp.sum(p, axis=1, keepdims=True)
        o_ref[:, sl] = _dot(a, v_ref[:, sl], (((1,), (0,)), ((), ())))


def _k3_proj(o_ref, x_ref, wo_ref, g2_ref, wg_ref, h_ref, xt_ref, lg_ref):
    hh = x_ref[...] + _dot(o_ref[...], wo_ref[...], (((1,), (0,)), ((), ())))
    xt = _rms(hh, g2_ref[...])
    h_ref[...] = hh
    xt_ref[...] = xt
    lg_ref[...] = _dot(xt, wg_ref[...], (((1,), (0,)), ((), ())))


def _k4_route(lg_ref, flat_ref, p_ref, aux_ref):
    CH = 128
    nch = S // CH
    r_i = jax.lax.broadcasted_iota(jnp.int32, (CH, CH), 0)
    c_i = jax.lax.broadcasted_iota(jnp.int32, (CH, CH), 1)
    tril = (r_i >= c_i).astype(jnp.float32)
    iota_e = jax.lax.broadcasted_iota(jnp.int32, (CH, E), 1)
    base = jnp.zeros((1, E), jnp.float32)
    p_acc = jnp.zeros((1, E), jnp.float32)
    lse2 = jnp.zeros((1, 1), jnp.float32)
    for c in range(nch):
        lg = lg_ref[c * CH:(c + 1) * CH, :]
        m = jnp.max(lg, axis=1, keepdims=True)
        ex = jnp.exp(lg - m)
        se = jnp.sum(ex, axis=1, keepdims=True)
        p_acc = p_acc + jnp.sum(ex / se, axis=0, keepdims=True)
        lse = m + jnp.log(se)
        lse2 = lse2 + jnp.sum(lse * lse, axis=0, keepdims=True)
        eidx = jnp.min(jnp.where(lg == m, iota_e, E), axis=1, keepdims=True)
        oh = (iota_e == eidx).astype(jnp.float32)
        csum = jax.lax.dot_general(tril, oh, (((1,), (0,)), ((), ())))
        pos = jnp.sum((csum + base) * oh, axis=1, keepdims=True).astype(
            jnp.int32) - 1
        keep = pos < CAP
        flat_ref[c * CH:(c + 1) * CH, :] = jnp.where(
            keep, eidx * CAP + pos, E * CAP)
        base = base + jnp.sum(oh, axis=0, keepdims=True)
    f = base * (1.0 / S)
    p = p_acc * (1.0 / S)
    p_ref[...] = p
    aux_ref[...] = (E * jnp.sum(f * p, keepdims=True).reshape(1, 1)
                    + ZW * lse2 * (1.0 / S))


def _k5_scatter(flat_ref, xt_ref, ein_ref):
    g = pl.program_id(0)

    @pl.when(g == 0)
    def _zero():
        ein_ref[...] = jnp.zeros((NPAD, D), jnp.float32)

    def body(i, _):
        idx = flat_ref[g * SBLK + i]
        ein_ref[pl.ds(idx, 1), :] = xt_ref[pl.ds(i, 1), :]
        return 0

    jax.lax.fori_loop(0, SBLK, body, 0)


def _k6_ffn(ein_ref, w1_ref, w2_ref, eout_ref):
    x = ein_ref[...]
    h1 = _dot(x, w1_ref[0], (((1,), (0,)), ((), ())))
    h1 = jax.nn.gelu(h1)
    eout_ref[...] = _dot(h1, w2_ref[0], (((1,), (0,)), ((), ())))


def _k7_gather(flat_ref, h_ref, eout_ref, out_ref):
    g = pl.program_id(0)
    out_ref[...] = h_ref[...]

    def body(i, _):
        idx = flat_ref[g * SBLK + i]
        safe = jnp.minimum(idx, E * CAP - 1)
        w = (idx < E * CAP).astype(jnp.float32)
        out_ref[pl.ds(i, 1), :] += eout_ref[pl.ds(safe, 1), :] * w
        return 0

    jax.lax.fori_loop(0, SBLK, body, 0)


@jax.jit
def kernel(x, ln1_g, ln2_g, Wqkv, Wo, Wg, W1, W2):
    xt2 = x.reshape(S, D)
    g1 = ln1_g.reshape(1, D)
    g2 = ln2_g.reshape(1, D)

    freqs = 1.0 / (10000.0 ** (jnp.arange(HALF, dtype=jnp.float32) / HALF))
    ang = jnp.arange(S, dtype=jnp.float32)[:, None] * freqs[None, :]
    cos_t = jnp.tile(jnp.concatenate([jnp.cos(ang)] * 2, axis=1), (1, H))
    sin_t = jnp.tile(
        jnp.concatenate([-jnp.sin(ang), jnp.sin(ang)], axis=1), (1, H))

    blk = lambda idx: pl.BlockSpec((SBLK, D), idx)
    q, k, v = pl.pallas_call(
        _k1_qkv,
        grid=(NS,),
        in_specs=[
            blk(lambda i: (i, 0)),
            pl.BlockSpec((1, D), lambda i: (0, 0)),
            pl.BlockSpec((D, 3 * D), lambda i: (0, 0)),
            blk(lambda i: (i, 0)),
            blk(lambda i: (i, 0)),
        ],
        out_specs=[blk(lambda i: (i, 0))] * 3,
        out_shape=[jax.ShapeDtypeStruct((S, D), jnp.float32)] * 3,
    )(xt2, g1, Wqkv, cos_t, sin_t)

    o = pl.pallas_call(
        _k2_attn,
        grid=(NS,),
        in_specs=[
            blk(lambda i: (i, 0)),
            pl.BlockSpec((S, D), lambda i: (0, 0)),
            pl.BlockSpec((S, D), lambda i: (0, 0)),
        ],
        out_specs=blk(lambda i: (i, 0)),
        out_shape=jax.ShapeDtypeStruct((S, D), jnp.float32),
    )(q, k, v)

    h, xt, lg = pl.pallas_call(
        _k3_proj,
        grid=(NS,),
        in_specs=[
            blk(lambda i: (i, 0)),
            blk(lambda i: (i, 0)),
            pl.BlockSpec((D, D), lambda i: (0, 0)),
            pl.BlockSpec((1, D), lambda i: (0, 0)),
            pl.BlockSpec((D, E), lambda i: (0, 0)),
        ],
        out_specs=[blk(lambda i: (i, 0)), blk(lambda i: (i, 0)),
                   pl.BlockSpec((SBLK, E), lambda i: (i, 0))],
        out_shape=[jax.ShapeDtypeStruct((S, D), jnp.float32),
                   jax.ShapeDtypeStruct((S, D), jnp.float32),
                   jax.ShapeDtypeStruct((S, E), jnp.float32)],
    )(o, xt2, Wo, g2, Wg)

    flat, p, aux = pl.pallas_call(
        _k4_route,
        in_specs=[pl.BlockSpec((S, E), lambda: (0, 0))],
        out_specs=[pl.BlockSpec((S, 1), lambda: (0, 0)),
                   pl.BlockSpec((1, E), lambda: (0, 0)),
                   pl.BlockSpec((1, 1), lambda: (0, 0))],
        out_shape=[jax.ShapeDtypeStruct((S, 1), jnp.int32),
                   jax.ShapeDtypeStruct((1, E), jnp.float32),
                   jax.ShapeDtypeStruct((1, 1), jnp.float32)],
    )(lg)

    flat1 = flat.reshape(S)

    ein = pl.pallas_call(
        _k5_scatter,
        grid_spec=pltpu.PrefetchScalarGridSpec(
            num_scalar_prefetch=1,
            grid=(NS,),
            in_specs=[pl.BlockSpec((SBLK, D), lambda i, s: (i, 0))],
            out_specs=pl.BlockSpec((NPAD, D), lambda i, s: (0, 0)),
        ),
        out_shape=jax.ShapeDtypeStruct((NPAD, D), jnp.float32),
    )(flat1, xt)

    eout = pl.pallas_call(
        _k6_ffn,
        grid=(E,),
        in_specs=[
            pl.BlockSpec((CAP, D), lambda e: (e, 0)),
            pl.BlockSpec((1, D, FF), lambda e: (e, 0, 0)),
            pl.BlockSpec((1, FF, D), lambda e: (e, 0, 0)),
        ],
        out_specs=pl.BlockSpec((CAP, D), lambda e: (e, 0)),
        out_shape=jax.ShapeDtypeStruct((E * CAP, D), jnp.float32),
    )(ein, W1, W2)

    out = pl.pallas_call(
        _k7_gather,
        grid_spec=pltpu.PrefetchScalarGridSpec(
            num_scalar_prefetch=1,
            grid=(NS,),
            in_specs=[pl.BlockSpec((SBLK, D), lambda i, s: (i, 0)),
                      pl.BlockSpec((E * CAP, D), lambda i, s: (0, 0))],
            out_specs=pl.BlockSpec((SBLK, D), lambda i, s: (i, 0)),
        ),
        out_shape=jax.ShapeDtypeStruct((S, D), jnp.float32),
    )(flat1, h, eout)

    return out.reshape(1, S, D), aux[0, 0], p.reshape(E)
